# baseline (speedup 1.0000x reference)
import functools

import jax
import jax.numpy as jnp
from jax import lax
from jax.experimental import pallas as pl
from jax.experimental.pallas import tpu as pltpu
from jax.experimental.pallas import tpu_sc as plsc

NUM_CENTERS = 65536
DIM = 128
NUM_DOCS = 1000000
BATCH = 4096


@functools.lru_cache(maxsize=None)
def _build():
    info = plsc.get_sparse_core_info()
    num_cores, num_subcores = info.num_cores, info.num_subcores
    num_workers = num_cores * num_subcores
    b_per_w = BATCH // num_workers

    mesh = plsc.VectorSubcoreMesh(core_axis_name="c", subcore_axis_name="s")

    n_chunks = 4
    cs = b_per_w // n_chunks

    @functools.partial(
        pl.kernel,
        mesh=mesh,
        out_type=jax.ShapeDtypeStruct((BATCH, DIM), jnp.float32),
        scratch_types=[
            pltpu.VMEM((b_per_w,), jnp.int32),
            pltpu.VMEM((b_per_w,), jnp.int32),
            pltpu.VMEM((b_per_w, DIM), jnp.float32),
            pltpu.SemaphoreType.DMA,
            pltpu.SemaphoreType.DMA,
            pltpu.SemaphoreType.DMA,
        ],
    )
    def sc_kernel(center_hbm, id2c_hbm, docid_hbm, out_hbm,
                  docid_v, cid_v, rows_v, sem_c, sem_r, sem_w):
        wid = lax.axis_index("s") * num_cores + lax.axis_index("c")
        base = wid * b_per_w
        pltpu.sync_copy(docid_hbm.at[pl.ds(base, b_per_w)], docid_v)
        cid_d = [None] * n_chunks
        row_d = [None] * n_chunks
        out_d = [None] * n_chunks
        cid_d[0] = pltpu.async_copy(
            id2c_hbm.at[docid_v.at[pl.ds(0, cs)]],
            cid_v.at[pl.ds(0, cs)], sem_c)
        for k in range(n_chunks):
            if k + 1 < n_chunks:
                cid_d[k + 1] = pltpu.async_copy(
                    id2c_hbm.at[docid_v.at[pl.ds((k + 1) * cs, cs)]],
                    cid_v.at[pl.ds((k + 1) * cs, cs)], sem_c)
            cid_d[k].wait()
            row_d[k] = pltpu.async_copy(
                center_hbm.at[cid_v.at[pl.ds(k * cs, cs)]],
                rows_v.at[pl.ds(k * cs, cs)], sem_r)
        for k in range(n_chunks):
            row_d[k].wait()
            out_d[k] = pltpu.async_copy(
                rows_v.at[pl.ds(k * cs, cs)],
                out_hbm.at[pl.ds(base + k * cs, cs)], sem_w)
        for k in range(n_chunks):
            out_d[k].wait()

    return sc_kernel


def kernel(center_vecs, id2center, doc_ids):
    return _build()(center_vecs, id2center, doc_ids)


# speedup vs baseline: 1.0294x; 1.0294x over previous
import functools

import jax
import jax.numpy as jnp
from jax import lax
from jax.experimental import pallas as pl
from jax.experimental.pallas import tpu as pltpu
from jax.experimental.pallas import tpu_sc as plsc

NUM_CENTERS = 65536
DIM = 128
NUM_DOCS = 1000000
BATCH = 4096


@functools.lru_cache(maxsize=None)
def _build():
    info = plsc.get_sparse_core_info()
    num_cores, num_subcores = info.num_cores, info.num_subcores
    num_workers = num_cores * num_subcores
    b_per_w = BATCH // num_workers

    mesh = plsc.VectorSubcoreMesh(core_axis_name="c", subcore_axis_name="s")

    n_chunks = 2
    cs = b_per_w // n_chunks

    @functools.partial(
        pl.kernel,
        mesh=mesh,
        out_type=jax.ShapeDtypeStruct((BATCH, DIM), jnp.float32),
        scratch_types=[
            pltpu.VMEM((b_per_w,), jnp.int32),
            pltpu.VMEM((b_per_w,), jnp.int32),
            pltpu.VMEM((b_per_w, DIM), jnp.float32),
            pltpu.SemaphoreType.DMA,
            pltpu.SemaphoreType.DMA,
            pltpu.SemaphoreType.DMA,
        ],
    )
    def sc_kernel(center_hbm, id2c_hbm, docid_hbm, out_hbm,
                  docid_v, cid_v, rows_v, sem_c, sem_r, sem_w):
        wid = lax.axis_index("s") * num_cores + lax.axis_index("c")
        base = wid * b_per_w
        pltpu.sync_copy(docid_hbm.at[pl.ds(base, b_per_w)], docid_v)
        cid_d = [None] * n_chunks
        row_d = [None] * n_chunks
        out_d = [None] * n_chunks
        cid_d[0] = pltpu.async_copy(
            id2c_hbm.at[docid_v.at[pl.ds(0, cs)]],
            cid_v.at[pl.ds(0, cs)], sem_c)
        for k in range(n_chunks):
            if k + 1 < n_chunks:
                cid_d[k + 1] = pltpu.async_copy(
                    id2c_hbm.at[docid_v.at[pl.ds((k + 1) * cs, cs)]],
                    cid_v.at[pl.ds((k + 1) * cs, cs)], sem_c)
            cid_d[k].wait()
            row_d[k] = pltpu.async_copy(
                center_hbm.at[cid_v.at[pl.ds(k * cs, cs)]],
                rows_v.at[pl.ds(k * cs, cs)], sem_r)
        for k in range(n_chunks):
            row_d[k].wait()
            out_d[k] = pltpu.async_copy(
                rows_v.at[pl.ds(k * cs, cs)],
                out_hbm.at[pl.ds(base + k * cs, cs)], sem_w)
        for k in range(n_chunks):
            out_d[k].wait()

    return sc_kernel


def kernel(center_vecs, id2center, doc_ids):
    return _build()(center_vecs, id2center, doc_ids)
